# baseline (speedup 1.0000x reference)
import functools

import jax
import jax.numpy as jnp
from jax import lax
from jax.experimental import pallas as pl
from jax.experimental.pallas import tpu as pltpu
from jax.experimental.pallas import tpu_sc as plsc

DIM = 32
B = 16384
NC, NS, L = 2, 16, 16
NW = NC * NS
BPW = B // NW
ICH = 128
CH = BPW // ICH

S3, S5, S7 = -1.0 / 6.0, 1.0 / 120.0, -1.0 / 5040.0
C2, C4, C6 = -0.5, 1.0 / 24.0, -1.0 / 720.0
RSQRT_MAGIC = 0x5F3759DF

_mesh = plsc.VectorSubcoreMesh(core_axis_name="c", subcore_axis_name="s")


@functools.partial(
    pl.kernel,
    out_type=jax.ShapeDtypeStruct((B,), jnp.float32),
    mesh=_mesh,
    compiler_params=pltpu.CompilerParams(
        needs_layout_passes=False, use_tc_tiling_on_sc=False
    ),
    scratch_types=[
        pltpu.VMEM((CH, ICH), jnp.int32),
        pltpu.VMEM((CH, ICH), jnp.int32),
        pltpu.VMEM((CH, ICH), jnp.int32),
        pltpu.VMEM((BPW, 2 * DIM), jnp.float32),
        pltpu.VMEM((BPW, 2 * DIM), jnp.float32),
        pltpu.VMEM((BPW, DIM), jnp.float32),
        pltpu.VMEM((BPW,), jnp.float32),
        pltpu.SemaphoreType.DMA,
        pltpu.SemaphoreType.DMA,
        pltpu.SemaphoreType.DMA,
    ],
)
def _rotate_sc(heads_hbm, rels_hbm, tails_hbm, ent_hbm, rel_hbm, out_hbm,
               hidx, ridx, tidx, hrows, trows, rrows, outv,
               sem_h, sem_t, sem_r):
    wid = lax.axis_index("s") * NC + lax.axis_index("c")
    base = wid * BPW

    pltpu.sync_copy(heads_hbm.at[wid], hidx)
    pltpu.sync_copy(tails_hbm.at[wid], tidx)
    pltpu.sync_copy(rels_hbm.at[wid], ridx)

    copies = []
    for j in range(CH):
        dst = pl.ds(j * ICH, ICH)
        copies.append(pltpu.async_copy(ent_hbm.at[hidx.at[j]], hrows.at[dst], sem_h))
        copies.append(pltpu.async_copy(ent_hbm.at[tidx.at[j]], trows.at[dst], sem_t))
        copies.append(pltpu.async_copy(rel_hbm.at[ridx.at[j]], rrows.at[dst], sem_r))
    for c in copies:
        c.wait()

    iota = lax.iota(jnp.int32, L)

    def group_body(g, carry):
        rows = g * L + iota

        def dim_body(d, acc):
            col = jnp.zeros((L,), jnp.int32) + d
            x = plsc.load_gather(rrows, [rows, col])
            hr = plsc.load_gather(hrows, [rows, col])
            hi = plsc.load_gather(hrows, [rows, col + DIM])
            tr = plsc.load_gather(trows, [rows, col])
            ti = plsc.load_gather(trows, [rows, col + DIM])
            x2 = x * x
            sn = x * (1.0 + x2 * (S3 + x2 * (S5 + x2 * S7)))
            cs = 1.0 + x2 * (C2 + x2 * (C4 + x2 * C6))
            dre = hr * cs - hi * sn - tr
            dim_ = hr * sn + hi * cs - ti
            m = dre * dre + dim_ * dim_
            hm = 0.5 * m
            yi = RSQRT_MAGIC - (plsc.bitcast(m, jnp.int32) >> 1)
            y = plsc.bitcast(yi, jnp.float32)
            y = y * (1.5 - (hm * y) * y)
            y = y * (1.5 - (hm * y) * y)
            y = y * (1.5 - (hm * y) * y)
            return acc + m * y

        acc = lax.fori_loop(0, DIM, dim_body, jnp.zeros((L,), jnp.float32))
        outv[pl.ds(g * L, L)] = acc
        return carry

    lax.fori_loop(0, BPW // L, group_body, 0)

    pltpu.sync_copy(outv, out_hbm.at[pl.ds(base, BPW)])


def kernel(heads, relations, tails, entity_table, relation_table):
    h = heads.astype(jnp.int32).reshape(NW, CH, ICH)
    t = tails.astype(jnp.int32).reshape(NW, CH, ICH)
    r = relations.astype(jnp.int32).reshape(NW, CH, ICH)
    return _rotate_sc(h, r, t, entity_table, relation_table)


# speedup vs baseline: 1.9489x; 1.9489x over previous
import functools

import jax
import jax.numpy as jnp
from jax import lax
from jax.experimental import pallas as pl
from jax.experimental.pallas import tpu as pltpu
from jax.experimental.pallas import tpu_sc as plsc

DIM = 32
B = 16384
NC, NS, L = 2, 16, 16
NW = NC * NS
BPW = B // NW
ICH = 128
CH = BPW // ICH
HALF = 2

S3, S5, S7 = -1.0 / 6.0, 1.0 / 120.0, -1.0 / 5040.0
C2, C4, C6 = -0.5, 1.0 / 24.0, -1.0 / 720.0
RSQRT_MAGIC = 0x5F3759DF

_mesh = plsc.VectorSubcoreMesh(core_axis_name="c", subcore_axis_name="s")

ECH = 8192
NEB = (1000000 + ECH - 1) // ECH


NROWS = NEB * (ECH // HALF)


def _detile_body(src_ref, out_ref):
    x = src_ref[...]
    r = lax.broadcasted_iota(jnp.int32, (2 * DIM, 2 * DIM), 0)
    c = lax.broadcasted_iota(jnp.int32, (2 * DIM, 2 * DIM), 1)
    eye = (r == c).astype(jnp.float32)
    y = lax.dot_general(x, eye, (((0,), (0,)), ((), ())),
                        preferred_element_type=jnp.float32)
    out_ref[...] = jnp.concatenate([y[: ECH // HALF], y[ECH // HALF :]], axis=1)


@functools.partial(
    pl.pallas_call,
    out_shape=jax.ShapeDtypeStruct((NROWS, 2 * DIM * HALF), jnp.float32),
    grid=(NEB,),
    in_specs=[pl.BlockSpec((2 * DIM, ECH), lambda b: (0, b))],
    out_specs=pl.BlockSpec((ECH // HALF, 2 * DIM * HALF), lambda b: (b, 0)),
    compiler_params=pltpu.CompilerParams(fuse_transposed_lhs_in_matmul=True),
)
def _detile_tc(src_ref, out_ref):
    _detile_body(src_ref, out_ref)


@functools.partial(
    pl.kernel,
    out_type=jax.ShapeDtypeStruct((B,), jnp.float32),
    mesh=_mesh,
    compiler_params=pltpu.CompilerParams(
        needs_layout_passes=False, use_tc_tiling_on_sc=False
    ),
    scratch_types=[
        pltpu.VMEM((CH, ICH), jnp.int32),
        pltpu.VMEM((CH, ICH), jnp.int32),
        pltpu.VMEM((CH, ICH), jnp.int32),
        pltpu.VMEM((BPW,), jnp.int32),
        pltpu.VMEM((BPW,), jnp.int32),
        pltpu.VMEM((BPW // 2, 2 * DIM * HALF), jnp.float32),
        pltpu.VMEM((BPW // 2, 2 * DIM * HALF), jnp.float32),
        pltpu.VMEM((BPW, DIM), jnp.float32),
        pltpu.VMEM((BPW,), jnp.float32),
        pltpu.SemaphoreType.DMA,
        pltpu.SemaphoreType.DMA,
        pltpu.SemaphoreType.DMA,
    ],
)
def _rotate_sc(hpair_hbm, rels_hbm, tpair_hbm, hpar_hbm, tpar_hbm,
               ent_hbm, rel_hbm, out_hbm,
               hidx, ridx, tidx, hpar, tpar, hrows, trows, rrows, outv,
               sem_h, sem_t, sem_r):
    wid = lax.axis_index("s") * NC + lax.axis_index("c")
    base = wid * BPW

    pltpu.sync_copy(hpair_hbm.at[wid], hidx)
    pltpu.sync_copy(tpair_hbm.at[wid], tidx)
    pltpu.sync_copy(rels_hbm.at[wid], ridx)
    pltpu.sync_copy(hpar_hbm.at[pl.ds(base, BPW)], hpar)
    pltpu.sync_copy(tpar_hbm.at[pl.ds(base, BPW)], tpar)

    iota = lax.iota(jnp.int32, L)
    rel_copies = []
    for j in range(CH):
        dst = pl.ds(j * ICH, ICH)
        rel_copies.append(
            pltpu.async_copy(rel_hbm.at[ridx.at[j]], rrows.at[dst], sem_r)
        )

    for p in range(2):
        ebase = p * (BPW // 2)
        copies = []
        for j in range(CH // 2):
            dst = pl.ds(j * ICH, ICH)
            src = p * (CH // 2) + j
            copies.append(
                pltpu.async_copy(ent_hbm.at[hidx.at[src]], hrows.at[dst], sem_h)
            )
            copies.append(
                pltpu.async_copy(ent_hbm.at[tidx.at[src]], trows.at[dst], sem_t)
            )
        for c in copies:
            c.wait()
        if p == 0:
            for c in rel_copies:
                c.wait()

        def group_body(g, carry):
            rows = g * L + iota
            hoff = hpar[pl.ds(ebase + g * L, L)]
            toff = tpar[pl.ds(ebase + g * L, L)]

            def dim_body(d, acc):
                col = jnp.zeros((L,), jnp.int32) + d
                x = plsc.load_gather(rrows, [ebase + rows, col])
                hr = plsc.load_gather(hrows, [rows, hoff + d])
                hi = plsc.load_gather(hrows, [rows, hoff + (d + DIM)])
                tr = plsc.load_gather(trows, [rows, toff + d])
                ti = plsc.load_gather(trows, [rows, toff + (d + DIM)])
                x2 = x * x
                sn = x * (1.0 + x2 * (S3 + x2 * (S5 + x2 * S7)))
                cs = 1.0 + x2 * (C2 + x2 * (C4 + x2 * C6))
                dre = hr * cs - hi * sn - tr
                dim_ = hr * sn + hi * cs - ti
                m = dre * dre + dim_ * dim_
                hm = 0.5 * m
                yi = RSQRT_MAGIC - (plsc.bitcast(m, jnp.int32) >> 1)
                y = plsc.bitcast(yi, jnp.float32)
                y = y * (1.5 - (hm * y) * y)
                y = y * (1.5 - (hm * y) * y)
                return acc + m * y

            acc = lax.fori_loop(0, DIM, dim_body, jnp.zeros((L,), jnp.float32))
            outv[pl.ds(ebase + g * L, L)] = acc
            return carry

        lax.fori_loop(0, BPW // (2 * L), group_body, 0)

    pltpu.sync_copy(outv, out_hbm.at[pl.ds(base, BPW)])


def kernel(heads, relations, tails, entity_table, relation_table):
    heads = heads.astype(jnp.int32)
    tails = tails.astype(jnp.int32)
    ent2 = _detile_tc(entity_table.T)
    hw = ECH // HALF

    def line_row(v):
        return (v // ECH) * hw + (v % hw)

    def line_par(v):
        return ((v % ECH) // hw) * (2 * DIM)

    h = line_row(heads).reshape(NW, CH, ICH)
    t = line_row(tails).reshape(NW, CH, ICH)
    hpar = line_par(heads)
    tpar = line_par(tails)
    r = relations.astype(jnp.int32).reshape(NW, CH, ICH)
    return _rotate_sc(h, r, t, hpar, tpar, ent2, relation_table)
